# baseline (speedup 1.0000x reference)
import functools

import jax
import jax.numpy as jnp
from jax import lax
from jax.experimental import pallas as pl
from jax.experimental.pallas import tpu as pltpu
from jax.experimental.pallas import tpu_sc as plsc

ROWS = 8192
COLS = 512
LANES = 16
CCH = COLS // LANES
NW = 32
RPW = ROWS // NW
CH = 64
NCH = RPW // CH
NSEG = 8


def _seg_partials_body(x_hbm, s_hbm, out_hbm, sbuf, xa, xb, accbuf,
                       sem_a, sem_b):
    cid = lax.axis_index("c")
    sid = lax.axis_index("s")
    wid = sid * 2 + cid
    lo = wid * RPW

    pltpu.sync_copy(s_hbm, sbuf)

    zero = jnp.zeros((LANES,), jnp.float32)
    for i in range(NSEG):
        accbuf[i, :] = zero

    svec = sbuf[...]
    s = [svec[i] for i in range(NSEG + 1)]

    bufs = [xa, xb]
    sems = [sem_a, sem_b]

    def start(c):
        return pltpu.async_copy(
            x_hbm.at[pl.ds(lo + c * CH, CH)], bufs[c % 2], sems[c % 2])

    def compute(c):
        buf = bufs[c % 2]
        r0 = lo + c * CH
        r1 = r0 + CH
        for i in range(NSEG):
            a = jnp.clip(s[i], r0, r1) - r0
            b = jnp.clip(s[i + 1], r0, r1) - r0

            def body(j, accs, buf=buf):
                accs = list(accs)
                for k in range(CCH):
                    accs[k % 4] = accs[k % 4] + buf[j, pl.ds(k * LANES, LANES)]
                return tuple(accs)

            a0, a1, a2, a3 = lax.fori_loop(a, b, body,
                                           (zero, zero, zero, zero))
            plsc.addupdate(accbuf.at[i], (a0 + a1) + (a2 + a3))

    descs = [start(0), start(1)]
    for c in range(NCH):
        descs[c].wait()
        compute(c)
        if c + 2 < NCH:
            descs.append(start(c + 2))

    pltpu.sync_copy(accbuf, out_hbm.at[wid])


@jax.jit
def _seg_partials(x, s16):
    mesh = plsc.VectorSubcoreMesh(
        core_axis_name="c", subcore_axis_name="s", num_cores=2,
        num_subcores=16)
    f = pl.kernel(
        _seg_partials_body,
        out_type=jax.ShapeDtypeStruct((NW, NSEG, LANES), jnp.float32),
        mesh=mesh,
        scratch_types=[
            pltpu.VMEM((LANES,), jnp.int32),
            pltpu.VMEM((CH, COLS), jnp.float32),
            pltpu.VMEM((CH, COLS), jnp.float32),
            pltpu.VMEM((NSEG, LANES), jnp.float32),
            pltpu.SemaphoreType.DMA,
            pltpu.SemaphoreType.DMA,
        ],
    )
    return f(x, s16)


def _combine_body(p_ref, o_ref):
    t = p_ref[...].reshape(NW, NSEG, LANES)
    g = jnp.sum(t, axis=0)
    sseg = jnp.sum(g, axis=1, keepdims=True)
    u = sseg[0:4] * sseg[4:8]
    v = u[0:2] * u[2:4]
    w = v[0:1] * v[1:2]
    o_ref[...] = w


def kernel(x, slices):
    s16 = jnp.pad(slices.astype(jnp.int32), (0, 7))
    partials = _seg_partials(x, s16)
    res = pl.pallas_call(
        _combine_body,
        out_shape=jax.ShapeDtypeStruct((1, 1), jnp.float32),
    )(partials.reshape(NW * NSEG, LANES))
    return res[0, 0]
